# baseline (speedup 1.0000x reference)
import jax
import jax.numpy as jnp
from jax.experimental import pallas as pl
from jax.experimental.pallas import tpu as pltpu

_SLOTS = 8
_TB = 512


def _moe_body(
    temp_ref, x_hbm, w_hbm, b_ref, o_hbm, xs_ref, os_ref, wv_ref, in_sems, out_sems, w_sems
):
    n = x_hbm.shape[0]
    e = w_hbm.shape[0]
    per = n // e
    chunks_per_expert = per // _TB
    nchunks = n // _TB

    def in_copy(i, slot):
        return pltpu.make_async_copy(
            x_hbm.at[pl.ds(i * _TB, _TB), :], xs_ref.at[slot], in_sems.at[slot]
        )

    def out_copy(i, slot):
        return pltpu.make_async_copy(
            os_ref.at[slot], o_hbm.at[pl.ds(i * _TB, _TB), :], out_sems.at[slot]
        )

    def w_copy(ei):
        return pltpu.make_async_copy(w_hbm.at[ei], wv_ref.at[ei], w_sems.at[ei])

    w_copy(0).start()
    for s in range(_SLOTS):
        in_copy(s, s).start()
    for ei in range(1, e):
        w_copy(ei).start()

    scale = jnp.exp(jnp.minimum(temp_ref[0, 0], jnp.log(jnp.float32(100.0))))

    def body(i, carry):
        slot = jax.lax.rem(i, _SLOTS)
        ei = jax.lax.div(i, chunks_per_expert)

        @pl.when(jax.lax.rem(i, chunks_per_expert) == 0)
        def _():
            w_copy(ei).wait()

        in_copy(i, slot).wait()
        acc = jax.lax.dot_general(
            xs_ref[slot],
            wv_ref[ei],
            (((1,), (1,)), ((), ())),
            preferred_element_type=jnp.float32,
        )

        @pl.when(i + _SLOTS < nchunks)
        def _():
            in_copy(i + _SLOTS, slot).start()

        @pl.when(i >= _SLOTS)
        def _():
            out_copy(i - _SLOTS, slot).wait()

        os_ref[slot] = (acc + b_ref[ei]) * scale
        out_copy(i, slot).start()
        return carry

    jax.lax.fori_loop(0, nchunks, body, 0)

    for s in range(_SLOTS):
        i = nchunks - _SLOTS + s
        out_copy(i, i % _SLOTS).wait()


def kernel(x, W, b, temperature):
    n, d = x.shape
    e = W.shape[0]
    temp2d = temperature.reshape(1, 1)

    out = pl.pallas_call(
        _moe_body,
        in_specs=[
            pl.BlockSpec(memory_space=pltpu.SMEM),
            pl.BlockSpec(memory_space=pl.ANY),
            pl.BlockSpec(memory_space=pl.ANY),
            pl.BlockSpec(memory_space=pltpu.VMEM),
        ],
        out_specs=pl.BlockSpec(memory_space=pl.ANY),
        out_shape=jax.ShapeDtypeStruct((n, d), x.dtype),
        scratch_shapes=[
            pltpu.VMEM((_SLOTS, _TB, d), jnp.float32),
            pltpu.VMEM((_SLOTS, _TB, d), jnp.float32),
            pltpu.VMEM((e, d, d), jnp.float32),
            pltpu.SemaphoreType.DMA((_SLOTS,)),
            pltpu.SemaphoreType.DMA((_SLOTS,)),
            pltpu.SemaphoreType.DMA((e,)),
        ],
        compiler_params=pltpu.CompilerParams(
            vmem_limit_bytes=100 * 1024 * 1024,
        ),
    )(temp2d, x, W, b.reshape(e, 1, d))

    aux_loss = jnp.float32(0.0)
    return (out, aux_loss)


# speedup vs baseline: 1.1187x; 1.1187x over previous
import jax
import jax.numpy as jnp
from jax.experimental import pallas as pl
from jax.experimental.pallas import tpu as pltpu

_SLOTS = 3
_TB = 2048


def _moe_body(
    temp_ref, x_hbm, w_hbm, b_ref, o_hbm, xs_ref, os_ref, wv_ref, in_sems, out_sems, w_sems
):
    n = x_hbm.shape[0]
    e = w_hbm.shape[0]
    per = n // e
    chunks_per_expert = per // _TB
    nchunks = n // _TB

    def in_copy(i, slot):
        return pltpu.make_async_copy(
            x_hbm.at[pl.ds(i * _TB, _TB), :], xs_ref.at[slot], in_sems.at[slot]
        )

    def out_copy(i, slot):
        return pltpu.make_async_copy(
            os_ref.at[slot], o_hbm.at[pl.ds(i * _TB, _TB), :], out_sems.at[slot]
        )

    def w_copy(ei):
        return pltpu.make_async_copy(w_hbm.at[ei], wv_ref.at[ei], w_sems.at[ei])

    w_copy(0).start()
    for s in range(_SLOTS):
        in_copy(s, s).start()
    for ei in range(1, e):
        w_copy(ei).start()

    scale = jnp.exp(jnp.minimum(temp_ref[0, 0], jnp.log(jnp.float32(100.0))))

    def body(i, carry):
        slot = jax.lax.rem(i, _SLOTS)
        ei = jax.lax.div(i, chunks_per_expert)

        @pl.when(jax.lax.rem(i, chunks_per_expert) == 0)
        def _():
            w_copy(ei).wait()

        in_copy(i, slot).wait()
        acc = jax.lax.dot_general(
            xs_ref[slot],
            wv_ref[ei],
            (((1,), (1,)), ((), ())),
            preferred_element_type=jnp.float32,
        )

        @pl.when(i + _SLOTS < nchunks)
        def _():
            in_copy(i + _SLOTS, slot).start()

        @pl.when(i >= _SLOTS)
        def _():
            out_copy(i - _SLOTS, slot).wait()

        os_ref[slot] = (acc + b_ref[ei]) * scale
        out_copy(i, slot).start()
        return carry

    jax.lax.fori_loop(0, nchunks, body, 0)

    for s in range(_SLOTS):
        i = nchunks - _SLOTS + s
        out_copy(i, i % _SLOTS).wait()


def kernel(x, W, b, temperature):
    n, d = x.shape
    e = W.shape[0]
    temp2d = temperature.reshape(1, 1)

    out = pl.pallas_call(
        _moe_body,
        in_specs=[
            pl.BlockSpec(memory_space=pltpu.SMEM),
            pl.BlockSpec(memory_space=pl.ANY),
            pl.BlockSpec(memory_space=pl.ANY),
            pl.BlockSpec(memory_space=pltpu.VMEM),
        ],
        out_specs=pl.BlockSpec(memory_space=pl.ANY),
        out_shape=jax.ShapeDtypeStruct((n, d), x.dtype),
        scratch_shapes=[
            pltpu.VMEM((_SLOTS, _TB, d), jnp.float32),
            pltpu.VMEM((_SLOTS, _TB, d), jnp.float32),
            pltpu.VMEM((e, d, d), jnp.float32),
            pltpu.SemaphoreType.DMA((_SLOTS,)),
            pltpu.SemaphoreType.DMA((_SLOTS,)),
            pltpu.SemaphoreType.DMA((e,)),
        ],
        compiler_params=pltpu.CompilerParams(
            vmem_limit_bytes=100 * 1024 * 1024,
        ),
    )(temp2d, x, W, b.reshape(e, 1, d))

    aux_loss = jnp.float32(0.0)
    return (out, aux_loss)


# speedup vs baseline: 1.1271x; 1.0076x over previous
import jax
import jax.numpy as jnp
from jax.experimental import pallas as pl
from jax.experimental.pallas import tpu as pltpu

_SLOTS = 4
_TB = 1024


def _moe_body(
    temp_ref, x_hbm, w_hbm, b_ref, o_hbm, xs_ref, os_ref, wv_ref, in_sems, out_sems, w_sems
):
    n = x_hbm.shape[0]
    e = w_hbm.shape[0]
    per = n // e
    chunks_per_expert = per // _TB
    nchunks = n // _TB

    def in_copy(i, slot):
        return pltpu.make_async_copy(
            x_hbm.at[pl.ds(i * _TB, _TB), :], xs_ref.at[slot], in_sems.at[slot]
        )

    def out_copy(i, slot):
        return pltpu.make_async_copy(
            os_ref.at[slot], o_hbm.at[pl.ds(i * _TB, _TB), :], out_sems.at[slot]
        )

    def w_copy(ei):
        return pltpu.make_async_copy(w_hbm.at[ei], wv_ref.at[ei], w_sems.at[ei])

    w_copy(0).start()
    for s in range(_SLOTS):
        in_copy(s, s).start()
    for ei in range(1, e):
        w_copy(ei).start()

    scale = jnp.exp(jnp.minimum(temp_ref[0, 0], jnp.log(jnp.float32(100.0))))

    def body(i, carry):
        slot = jax.lax.rem(i, _SLOTS)
        ei = jax.lax.div(i, chunks_per_expert)

        @pl.when(jax.lax.rem(i, chunks_per_expert) == 0)
        def _():
            w_copy(ei).wait()

        in_copy(i, slot).wait()
        acc = jax.lax.dot_general(
            xs_ref[slot],
            wv_ref[ei],
            (((1,), (1,)), ((), ())),
            preferred_element_type=jnp.float32,
        )

        @pl.when(i + _SLOTS < nchunks)
        def _():
            in_copy(i + _SLOTS, slot).start()

        @pl.when(i >= _SLOTS)
        def _():
            out_copy(i - _SLOTS, slot).wait()

        os_ref[slot] = (acc + b_ref[ei]) * scale
        out_copy(i, slot).start()
        return carry

    jax.lax.fori_loop(0, nchunks, body, 0)

    for s in range(_SLOTS):
        i = nchunks - _SLOTS + s
        out_copy(i, i % _SLOTS).wait()


def kernel(x, W, b, temperature):
    n, d = x.shape
    e = W.shape[0]
    temp2d = temperature.reshape(1, 1)

    out = pl.pallas_call(
        _moe_body,
        in_specs=[
            pl.BlockSpec(memory_space=pltpu.SMEM),
            pl.BlockSpec(memory_space=pl.ANY),
            pl.BlockSpec(memory_space=pl.ANY),
            pl.BlockSpec(memory_space=pltpu.VMEM),
        ],
        out_specs=pl.BlockSpec(memory_space=pl.ANY),
        out_shape=jax.ShapeDtypeStruct((n, d), x.dtype),
        scratch_shapes=[
            pltpu.VMEM((_SLOTS, _TB, d), jnp.float32),
            pltpu.VMEM((_SLOTS, _TB, d), jnp.float32),
            pltpu.VMEM((e, d, d), jnp.float32),
            pltpu.SemaphoreType.DMA((_SLOTS,)),
            pltpu.SemaphoreType.DMA((_SLOTS,)),
            pltpu.SemaphoreType.DMA((e,)),
        ],
        compiler_params=pltpu.CompilerParams(
            vmem_limit_bytes=100 * 1024 * 1024,
        ),
    )(temp2d, x, W, b.reshape(e, 1, d))

    aux_loss = jnp.float32(0.0)
    return (out, aux_loss)
